# Initial kernel scaffold; baseline (speedup 1.0000x reference)
import jax
import jax.numpy as jnp
from jax.experimental import pallas as pl


def kernel(x, edge_index, timestep, W_in, b_in, g_in, be_in, Wtm, btm, Wt1, bt1, W1, b1, g1, be1, Wt2, bt2, W2, b2, g2, be2, W_out, b_out, g_out, be_out):
    raise NotImplementedError("write your pallas kernel here")


# speedup vs baseline: 6.9802x; 6.9802x over previous
import functools
import math

import jax
import jax.numpy as jnp
from jax import lax
from jax.experimental import pallas as pl
from jax.experimental.pallas import tpu as pltpu
from jax.experimental.pallas import tpu_sc as plsc

N = 10000
NP = 10240
E = 320000
D = 128
H1 = 384
H2 = 1152
T = 128

NC, NS = 2, 16
CH = 128
EP = 323584
BR = 640
NRB = NP // BR
EPS = 1e-5

_SC_MESH = plsc.VectorSubcoreMesh(
    core_axis_name="c", subcore_axis_name="s", num_cores=NC, num_subcores=NS)


def _deg_body(dst_hbm, ones_hbm, zeros_hbm, out_hbm, idx_v, ones_v, buf_v,
              deg_sh):
    c = lax.axis_index("c")
    s = lax.axis_index("s")
    epw = EP // (NC * NS)
    nch = epw // CH
    nzr = NP // NS
    pltpu.sync_copy(zeros_hbm.at[pl.ds(s * nzr, nzr)],
                    deg_sh.at[pl.ds(s * nzr, nzr)])
    pltpu.sync_copy(ones_hbm, ones_v)
    plsc.subcore_barrier()
    base = c * (EP // NC) + s * epw

    def chunk(i, carry):
        pltpu.sync_copy(dst_hbm.at[pl.ds(base + i * CH, CH)], idx_v)
        pltpu.sync_copy(ones_v, deg_sh.at[idx_v], add=True)
        return carry

    lax.fori_loop(0, nch, chunk, 0, unroll=False)
    plsc.subcore_barrier()
    pltpu.sync_copy(deg_sh.at[pl.ds(s * nzr, nzr)], buf_v)
    pltpu.sync_copy(buf_v, out_hbm.at[c, pl.ds(s * nzr, nzr)])


_deg_kernel = pl.kernel(
    _deg_body,
    out_type=jax.ShapeDtypeStruct((NC, NP), jnp.float32),
    mesh=_SC_MESH,
    scratch_types=[
        pltpu.VMEM((CH,), jnp.int32),
        pltpu.VMEM((CH,), jnp.float32),
        pltpu.VMEM((NP // NS,), jnp.float32),
        pltpu.VMEM_SHARED((NP,), jnp.float32),
    ],
)


def _scatter_pass(c, s, slab, ebase, nch, oslot, src_hbm, dst_hbm, xs_hbm,
                  zeros_hbm, out_hbm, idxs_v, idxd_v, rows_v, acc_sh):
    nzr = NP // NS
    pltpu.sync_copy(zeros_hbm.at[pl.ds(s * nzr, nzr), :],
                    acc_sh.at[pl.ds(s * nzr, nzr), :])
    plsc.subcore_barrier()
    base = ebase + s * nch * CH

    def chunk(i, carry):
        off = base + i * CH
        pltpu.sync_copy(src_hbm.at[pl.ds(off, CH)], idxs_v)
        pltpu.sync_copy(dst_hbm.at[pl.ds(off, CH)], idxd_v)
        pltpu.sync_copy(xs_hbm.at[slab].at[idxs_v], rows_v)
        pltpu.sync_copy(rows_v, acc_sh.at[idxd_v], add=True)
        return carry

    lax.fori_loop(0, nch, chunk, 0, unroll=False)
    plsc.subcore_barrier()
    for j in range(nzr // CH):
        r0 = s * nzr + j * CH
        pltpu.sync_copy(acc_sh.at[pl.ds(r0, CH), :], rows_v)
        pltpu.sync_copy(rows_v, out_hbm.at[oslot].at[pl.ds(r0, CH), :])
    plsc.subcore_barrier()


def _prop128_body(src_hbm, dst_hbm, xs_hbm, zeros_hbm, out_hbm,
                  idxs_v, idxd_v, rows_v, acc_sh):
    c = lax.axis_index("c")
    s = lax.axis_index("s")
    nch = EP // (NC * NS * CH)
    _scatter_pass(c, s, 0, c * (EP // NC), nch, c,
                  src_hbm, dst_hbm, xs_hbm, zeros_hbm, out_hbm,
                  idxs_v, idxd_v, rows_v, acc_sh)


def _prop384_body(src_hbm, dst_hbm, xs_hbm, zeros_hbm, out_hbm,
                  idxs_v, idxd_v, rows_v, acc_sh):
    c = lax.axis_index("c")
    s = lax.axis_index("s")
    nch0 = EP // (NS * CH)
    _scatter_pass(c, s, c, 0, nch0, c,
                  src_hbm, dst_hbm, xs_hbm, zeros_hbm, out_hbm,
                  idxs_v, idxd_v, rows_v, acc_sh)
    nch1 = EP // (NC * NS * CH)
    _scatter_pass(c, s, 2, c * (EP // NC), nch1, 2 + c,
                  src_hbm, dst_hbm, xs_hbm, zeros_hbm, out_hbm,
                  idxs_v, idxd_v, rows_v, acc_sh)


def _make_prop(body, nslab_in, nslab_out):
    return pl.kernel(
        body,
        out_type=jax.ShapeDtypeStruct((nslab_out, NP, CH), jnp.float32),
        mesh=_SC_MESH,
        scratch_types=[
            pltpu.VMEM((CH,), jnp.int32),
            pltpu.VMEM((CH,), jnp.int32),
            pltpu.VMEM((CH, CH), jnp.float32),
            pltpu.VMEM_SHARED((NP, CH), jnp.float32),
        ],
    )


_prop128 = _make_prop(_prop128_body, 1, 2)
_prop384 = _make_prop(_prop384_body, 3, 4)


def _rowmask(pid):
    rows = lax.broadcasted_iota(jnp.int32, (BR, 1), 0) + pid * BR
    return rows < N


def _stats(ps_ref, pq_ref, r, pid):
    rm = jnp.where(_rowmask(pid), r, 0.0)
    ps_ref[...] = jnp.sum(rm, axis=0, keepdims=True)[None]
    pq_ref[...] = jnp.sum(rm * rm, axis=0, keepdims=True)[None]


def _bn_coeff(ps_ref, pq_ref, g_ref, be_ref):
    m = jnp.sum(ps_ref[...], axis=0) / N
    mq = jnp.sum(pq_ref[...], axis=0) / N
    v = mq - m * m
    a = g_ref[...] / jnp.sqrt(v + EPS)
    return a, be_ref[...] - m * a


def _k0_body(deg_ref, x_ref, ts_ref, wtm_ref, btm_ref, wt1_ref, bt1_ref,
             wt2_ref, bt2_ref, dis_ref, xs0_ref, te1_ref, te2_ref):
    pid = pl.program_id(0)
    deg = deg_ref[0, :] + deg_ref[1, :] + 1.0
    dis = (1.0 / jnp.sqrt(deg))[:, None]
    dis_ref[...] = dis
    xs0_ref[0] = x_ref[...] * dis

    @pl.when(pid == 0)
    def _():
        half = T // 2
        k = math.log(10000.0) / (half - 1)
        e = jnp.exp(
            lax.broadcasted_iota(jnp.int32, (1, half), 1).astype(jnp.float32)
            * (-k))
        e = ts_ref[0, 0] * e
        emb = jnp.concatenate([jnp.sin(e), jnp.cos(e)], axis=-1)
        t = jnp.maximum(
            jnp.dot(emb, wtm_ref[...], preferred_element_type=jnp.float32)
            + btm_ref[...], 0.0)
        te1_ref[...] = jnp.maximum(
            jnp.dot(t, wt1_ref[...], preferred_element_type=jnp.float32)
            + bt1_ref[...], 0.0)
        te2_ref[...] = jnp.maximum(
            jnp.dot(t, wt2_ref[...], preferred_element_type=jnp.float32)
            + bt2_ref[...], 0.0)


def _k1_body(acc_ref, xs_ref, dis_ref, w_ref, b_ref, u_ref, ps_ref, pq_ref):
    pid = pl.program_id(0)
    g = dis_ref[...] * (acc_ref[0] + acc_ref[1] + xs_ref[0])
    u = jnp.dot(g, w_ref[...], preferred_element_type=jnp.float32) + b_ref[...]
    u_ref[...] = u
    _stats(ps_ref, pq_ref, u, pid)


def _k2_body(u_ref, ps_ref, pq_ref, g_ref, be_ref, dis_ref, xs_ref):
    a, cb = _bn_coeff(ps_ref, pq_ref, g_ref, be_ref)
    h = jnp.maximum(a * u_ref[...] + cb, 0.0)
    xs = dis_ref[...] * h
    xs_ref[...] = jnp.stack([xs[:, :128], xs[:, 128:256], xs[:, 256:]])


def _k3_body(acc_ref, xs_ref, dis_ref, w_ref, b_ref, te_ref,
             r_ref, ps_ref, pq_ref):
    pid = pl.program_id(0)
    dis = dis_ref[...]
    p = jnp.concatenate(
        [acc_ref[0] + xs_ref[0], acc_ref[1] + xs_ref[1],
         acc_ref[2] + acc_ref[3] + xs_ref[2]], axis=1) * dis
    u = jnp.dot(p, w_ref[...], preferred_element_type=jnp.float32) + b_ref[...]
    r = jnp.maximum(u + te_ref[...], 0.0)
    r_ref[...] = r
    _stats(ps_ref, pq_ref, r, pid)


def _k4_body(r_ref, ps_ref, pq_ref, g_ref, be_ref, w_ref, dis_ref, xs_ref):
    a, cb = _bn_coeff(ps_ref, pq_ref, g_ref, be_ref)
    h = a * r_ref[...] + cb
    xs = dis_ref[...] * jnp.dot(
        h, w_ref[...], preferred_element_type=jnp.float32)
    xs_ref[...] = jnp.stack([xs[:, :128], xs[:, 128:256], xs[:, 256:]])


def _k5_body(acc_ref, xs_ref, dis_ref, b_ref, te_ref, r_ref, ps_ref, pq_ref):
    pid = pl.program_id(0)
    dis = dis_ref[...]
    w = jnp.concatenate(
        [acc_ref[0] + xs_ref[0], acc_ref[1] + xs_ref[1],
         acc_ref[2] + acc_ref[3] + xs_ref[2]], axis=1) * dis + b_ref[...]
    r = jnp.maximum(w + te_ref[...], 0.0)
    r_ref[...] = r
    _stats(ps_ref, pq_ref, r, pid)


def _k6_body(r_ref, ps_ref, pq_ref, g_ref, be_ref, w_ref, dis_ref, xs_ref):
    a, cb = _bn_coeff(ps_ref, pq_ref, g_ref, be_ref)
    h = a * r_ref[...] + cb
    xs_ref[0] = dis_ref[...] * jnp.dot(
        h, w_ref[...], preferred_element_type=jnp.float32)


def _k7_body(acc_ref, xs_ref, dis_ref, b_ref, o_ref, ps_ref, pq_ref):
    pid = pl.program_id(0)
    o = (dis_ref[...] * (acc_ref[0] + acc_ref[1] + xs_ref[0]) + b_ref[...])
    o_ref[...] = o
    _stats(ps_ref, pq_ref, o, pid)


def _k8_body(o_ref, ps_ref, pq_ref, g_ref, be_ref, out_ref):
    a, cb = _bn_coeff(ps_ref, pq_ref, g_ref, be_ref)
    out_ref[...] = jnp.maximum(a * o_ref[...] + cb, 0.0)


def _vmem(block, index_map):
    return pl.BlockSpec(block, index_map)


def kernel(x, edge_index, timestep, W_in, b_in, g_in, be_in, Wtm, btm,
           Wt1, bt1, W1, b1, g1, be1, Wt2, bt2, W2, b2, g2, be2,
           W_out, b_out, g_out, be_out):
    f32 = jnp.float32
    src = jnp.concatenate(
        [edge_index[0].astype(jnp.int32),
         jnp.zeros((EP - E,), jnp.int32)])
    dst = jnp.concatenate(
        [edge_index[1].astype(jnp.int32),
         jnp.full((EP - E,), N, jnp.int32)])
    xp = jnp.pad(x, ((0, NP - N), (0, 0)))
    zeros_r = jnp.zeros((NP,), f32)
    zeros_s = jnp.zeros((NP, CH), f32)
    ones_ch = jnp.ones((CH,), f32)
    ts = timestep.reshape(1, 1)
    b_in2 = b_in.reshape(1, H1)
    g_in2 = g_in.reshape(1, H1)
    be_in2 = be_in.reshape(1, H1)
    b1_2 = b1.reshape(1, H2)
    g1_2 = g1.reshape(1, H2)
    be1_2 = be1.reshape(1, H2)
    b2_2 = b2.reshape(1, H1)
    g2_2 = g2.reshape(1, H1)
    be2_2 = be2.reshape(1, H1)
    bo_2 = b_out.reshape(1, D)
    go_2 = g_out.reshape(1, D)
    beo_2 = be_out.reshape(1, D)

    deg2 = _deg_kernel(dst, ones_ch, zeros_r)

    dis, xs0, te1, te2 = pl.pallas_call(
        _k0_body,
        grid=(NRB,),
        in_specs=[
            _vmem((NC, BR), lambda r: (0, r)),
            _vmem((BR, D), lambda r: (r, 0)),
            _vmem((1, 1), lambda r: (0, 0)),
            _vmem((T, T), lambda r: (0, 0)),
            _vmem((1, T), lambda r: (0, 0)),
            _vmem((T, H2), lambda r: (0, 0)),
            _vmem((1, H2), lambda r: (0, 0)),
            _vmem((T, H1), lambda r: (0, 0)),
            _vmem((1, H1), lambda r: (0, 0)),
        ],
        out_specs=[
            _vmem((BR, 1), lambda r: (r, 0)),
            _vmem((1, BR, D), lambda r: (0, r, 0)),
            _vmem((1, H2), lambda r: (0, 0)),
            _vmem((1, H1), lambda r: (0, 0)),
        ],
        out_shape=[
            jax.ShapeDtypeStruct((NP, 1), f32),
            jax.ShapeDtypeStruct((1, NP, D), f32),
            jax.ShapeDtypeStruct((1, H2), f32),
            jax.ShapeDtypeStruct((1, H1), f32),
        ],
    )(deg2, xp, ts, Wtm, btm.reshape(1, T), Wt1, bt1.reshape(1, H2),
      Wt2, bt2.reshape(1, H1))

    acc0 = _prop128(src, dst, xs0, zeros_s)

    u, ps, pq = pl.pallas_call(
        _k1_body,
        grid=(NRB,),
        in_specs=[
            _vmem((2, BR, D), lambda r: (0, r, 0)),
            _vmem((1, BR, D), lambda r: (0, r, 0)),
            _vmem((BR, 1), lambda r: (r, 0)),
            _vmem((D, H1), lambda r: (0, 0)),
            _vmem((1, H1), lambda r: (0, 0)),
        ],
        out_specs=[
            _vmem((BR, H1), lambda r: (r, 0)),
            _vmem((1, 1, H1), lambda r: (r, 0, 0)),
            _vmem((1, 1, H1), lambda r: (r, 0, 0)),
        ],
        out_shape=[
            jax.ShapeDtypeStruct((NP, H1), f32),
            jax.ShapeDtypeStruct((NRB, 1, H1), f32),
            jax.ShapeDtypeStruct((NRB, 1, H1), f32),
        ],
    )(acc0, xs0, dis, W_in, b_in2)

    xs1 = pl.pallas_call(
        _k2_body,
        grid=(NRB,),
        in_specs=[
            _vmem((BR, H1), lambda r: (r, 0)),
            _vmem((NRB, 1, H1), lambda r: (0, 0, 0)),
            _vmem((NRB, 1, H1), lambda r: (0, 0, 0)),
            _vmem((1, H1), lambda r: (0, 0)),
            _vmem((1, H1), lambda r: (0, 0)),
            _vmem((BR, 1), lambda r: (r, 0)),
        ],
        out_specs=_vmem((3, BR, CH), lambda r: (0, r, 0)),
        out_shape=jax.ShapeDtypeStruct((3, NP, CH), f32),
    )(u, ps, pq, g_in2, be_in2, dis)

    acc1 = _prop384(src, dst, xs1, zeros_s)

    r1, ps, pq = pl.pallas_call(
        _k3_body,
        grid=(NRB,),
        in_specs=[
            _vmem((4, BR, CH), lambda r: (0, r, 0)),
            _vmem((3, BR, CH), lambda r: (0, r, 0)),
            _vmem((BR, 1), lambda r: (r, 0)),
            _vmem((H1, H2), lambda r: (0, 0)),
            _vmem((1, H2), lambda r: (0, 0)),
            _vmem((1, H2), lambda r: (0, 0)),
        ],
        out_specs=[
            _vmem((BR, H2), lambda r: (r, 0)),
            _vmem((1, 1, H2), lambda r: (r, 0, 0)),
            _vmem((1, 1, H2), lambda r: (r, 0, 0)),
        ],
        out_shape=[
            jax.ShapeDtypeStruct((NP, H2), f32),
            jax.ShapeDtypeStruct((NRB, 1, H2), f32),
            jax.ShapeDtypeStruct((NRB, 1, H2), f32),
        ],
    )(acc1, xs1, dis, W1, b1_2, te1)

    xs2 = pl.pallas_call(
        _k4_body,
        grid=(NRB,),
        in_specs=[
            _vmem((BR, H2), lambda r: (r, 0)),
            _vmem((NRB, 1, H2), lambda r: (0, 0, 0)),
            _vmem((NRB, 1, H2), lambda r: (0, 0, 0)),
            _vmem((1, H2), lambda r: (0, 0)),
            _vmem((1, H2), lambda r: (0, 0)),
            _vmem((H2, H1), lambda r: (0, 0)),
            _vmem((BR, 1), lambda r: (r, 0)),
        ],
        out_specs=_vmem((3, BR, CH), lambda r: (0, r, 0)),
        out_shape=jax.ShapeDtypeStruct((3, NP, CH), f32),
    )(r1, ps, pq, g1_2, be1_2, W2, dis)

    acc2 = _prop384(src, dst, xs2, zeros_s)

    r2, ps, pq = pl.pallas_call(
        _k5_body,
        grid=(NRB,),
        in_specs=[
            _vmem((4, BR, CH), lambda r: (0, r, 0)),
            _vmem((3, BR, CH), lambda r: (0, r, 0)),
            _vmem((BR, 1), lambda r: (r, 0)),
            _vmem((1, H1), lambda r: (0, 0)),
            _vmem((1, H1), lambda r: (0, 0)),
        ],
        out_specs=[
            _vmem((BR, H1), lambda r: (r, 0)),
            _vmem((1, 1, H1), lambda r: (r, 0, 0)),
            _vmem((1, 1, H1), lambda r: (r, 0, 0)),
        ],
        out_shape=[
            jax.ShapeDtypeStruct((NP, H1), f32),
            jax.ShapeDtypeStruct((NRB, 1, H1), f32),
            jax.ShapeDtypeStruct((NRB, 1, H1), f32),
        ],
    )(acc2, xs2, dis, b2_2, te2)

    xs3 = pl.pallas_call(
        _k6_body,
        grid=(NRB,),
        in_specs=[
            _vmem((BR, H1), lambda r: (r, 0)),
            _vmem((NRB, 1, H1), lambda r: (0, 0, 0)),
            _vmem((NRB, 1, H1), lambda r: (0, 0, 0)),
            _vmem((1, H1), lambda r: (0, 0)),
            _vmem((1, H1), lambda r: (0, 0)),
            _vmem((H1, D), lambda r: (0, 0)),
            _vmem((BR, 1), lambda r: (r, 0)),
        ],
        out_specs=_vmem((1, BR, D), lambda r: (0, r, 0)),
        out_shape=jax.ShapeDtypeStruct((1, NP, D), f32),
    )(r2, ps, pq, g2_2, be2_2, W_out, dis)

    acc3 = _prop128(src, dst, xs3, zeros_s)

    o, ps, pq = pl.pallas_call(
        _k7_body,
        grid=(NRB,),
        in_specs=[
            _vmem((2, BR, D), lambda r: (0, r, 0)),
            _vmem((1, BR, D), lambda r: (0, r, 0)),
            _vmem((BR, 1), lambda r: (r, 0)),
            _vmem((1, D), lambda r: (0, 0)),
        ],
        out_specs=[
            _vmem((BR, D), lambda r: (r, 0)),
            _vmem((1, 1, D), lambda r: (r, 0, 0)),
            _vmem((1, 1, D), lambda r: (r, 0, 0)),
        ],
        out_shape=[
            jax.ShapeDtypeStruct((NP, D), f32),
            jax.ShapeDtypeStruct((NRB, 1, D), f32),
            jax.ShapeDtypeStruct((NRB, 1, D), f32),
        ],
    )(acc3, xs3, dis, bo_2)

    out = pl.pallas_call(
        _k8_body,
        grid=(NRB,),
        in_specs=[
            _vmem((BR, D), lambda r: (r, 0)),
            _vmem((NRB, 1, D), lambda r: (0, 0, 0)),
            _vmem((NRB, 1, D), lambda r: (0, 0, 0)),
            _vmem((1, D), lambda r: (0, 0)),
            _vmem((1, D), lambda r: (0, 0)),
        ],
        out_specs=_vmem((BR, D), lambda r: (r, 0)),
        out_shape=jax.ShapeDtypeStruct((NP, D), f32),
    )(o, ps, pq, go_2, beo_2)

    return out[:N]
